# Initial kernel scaffold; baseline (speedup 1.0000x reference)
import jax
import jax.numpy as jnp
from jax.experimental import pallas as pl


def kernel(x, edge_index, params):
    raise NotImplementedError("write your pallas kernel here")


# speedup vs baseline: 1.3756x; 1.3756x over previous
import functools

import jax
import jax.numpy as jnp
from jax import lax
from jax.experimental import pallas as pl
from jax.experimental.pallas import tpu as pltpu
from jax.experimental.pallas import tpu_sc as plsc

N = 10000
E = 160000
D_IN = 256
HID = 512
D_OUT = 512
D_VAE = 128
L = 4

NC = 2
NS = 16
NW = NC * NS
CHUNK = 128
NCHUNK = 40
EPW = CHUNK * NCHUNK
PE = EPW * NW
DUMMY = N
RPT = 640
R = RPT * NS

BN = 400
NB = N // BN


@functools.cache
def _make_sc_agg(C):
    mesh = plsc.VectorSubcoreMesh(core_axis_name="c", subcore_axis_name="s")

    @functools.partial(
        pl.kernel,
        out_type=jax.ShapeDtypeStruct((NC, C, N, 128), jnp.float32),
        mesh=mesh,
        scratch_types=[
            pltpu.VMEM((CHUNK,), jnp.int32),
            pltpu.VMEM((CHUNK,), jnp.int32),
            pltpu.VMEM((CHUNK, 128), jnp.float32),
            pltpu.VMEM((16, 128), jnp.float32),
            pltpu.VMEM_SHARED((R, 128), jnp.float32),
            pltpu.SemaphoreType.DMA,
        ],
    )
    def sc_agg(hcb, srcp, dstp, out, src_v, dst_v, rows_v, zb_v, acc, sem):
        cid = lax.axis_index("c")
        sid = lax.axis_index("s")
        base = (sid * NC + cid) * EPW
        row0 = sid * RPT

        zeros16 = jnp.zeros((16,), jnp.float32)
        for i in range(16):
            for j in range(8):
                zb_v[i, pl.ds(j * 16, 16)] = zeros16

        for cc in range(C):
            def zero_body(k, carry):
                pltpu.sync_copy(zb_v, acc.at[pl.ds(row0 + k * 16, 16)])
                return carry
            lax.fori_loop(0, RPT // 16, zero_body, 0)
            plsc.subcore_barrier()

            def edge_body(t, carry):
                off = base + t * CHUNK
                pltpu.sync_copy(srcp.at[pl.ds(off, CHUNK)], src_v)
                pltpu.sync_copy(dstp.at[pl.ds(off, CHUNK)], dst_v)
                pltpu.async_copy(hcb.at[cc].at[src_v], rows_v, sem).wait()
                pltpu.sync_copy(rows_v, acc.at[dst_v], add=True)
                return carry
            lax.fori_loop(0, NCHUNK, edge_body, 0)
            plsc.subcore_barrier()

            @pl.when(sid < NS - 1)
            def _():
                pltpu.sync_copy(acc.at[pl.ds(row0, RPT)],
                                out.at[cid, cc, pl.ds(row0, RPT)])

            @pl.when(sid == NS - 1)
            def _():
                pltpu.sync_copy(acc.at[pl.ds(row0, N - (NS - 1) * RPT)],
                                out.at[cid, cc, pl.ds(row0, N - (NS - 1) * RPT)])

    return sc_agg


@functools.cache
def _make_mm(C):
    K = C * 128

    def body(eps_ref, h_ref, p_ref, w1_ref, b1_ref, w2_ref, b2_ref,
             y_ref, st_ref):
        eps1 = 1.0 + eps_ref[0]
        t = jnp.zeros((BN, HID), jnp.float32)
        for c in range(C):
            zc = eps1 * h_ref[c] + p_ref[0, c] + p_ref[1, c]
            t = t + jnp.dot(zc.astype(jnp.bfloat16),
                            w1_ref[c * 128:(c + 1) * 128, :].astype(jnp.bfloat16),
                            preferred_element_type=jnp.float32)
        y1 = jnp.maximum(t + b1_ref[...], 0.0)
        y = jnp.dot(y1.astype(jnp.bfloat16), w2_ref[...].astype(jnp.bfloat16),
                    preferred_element_type=jnp.float32) + b2_ref[...]
        y_ref[...] = y

        @pl.when(pl.program_id(0) == 0)
        def _():
            st_ref[...] = jnp.zeros_like(st_ref)
        st_ref[...] += jnp.concatenate(
            [jnp.sum(y, 0, keepdims=True), jnp.sum(y * y, 0, keepdims=True)], 0)

    return pl.pallas_call(
        body,
        grid=(NB,),
        in_specs=[
            pl.BlockSpec(memory_space=pltpu.SMEM),
            pl.BlockSpec((C, BN, 128), lambda i: (0, i, 0)),
            pl.BlockSpec((NC, C, BN, 128), lambda i: (0, 0, i, 0)),
            pl.BlockSpec((K, HID), lambda i: (0, 0)),
            pl.BlockSpec((1, HID), lambda i: (0, 0)),
            pl.BlockSpec((HID, HID), lambda i: (0, 0)),
            pl.BlockSpec((1, HID), lambda i: (0, 0)),
        ],
        out_specs=[
            pl.BlockSpec((BN, HID), lambda i: (i, 0)),
            pl.BlockSpec((2, HID), lambda i: (0, 0)),
        ],
        out_shape=[
            jax.ShapeDtypeStruct((N, HID), jnp.float32),
            jax.ShapeDtypeStruct((2, HID), jnp.float32),
        ],
    )


def _bn_body(y_ref, st_ref, g_ref, b_ref, o_ref):
    mu = st_ref[0:1, :] * (1.0 / N)
    ex2 = st_ref[1:2, :] * (1.0 / N)
    var = ex2 - mu * mu
    inv = lax.rsqrt(var + 1e-5)
    o_ref[0] = jnp.maximum((y_ref[...] - mu) * inv * g_ref[...] + b_ref[...],
                           0.0)


_bn_apply = pl.pallas_call(
    _bn_body,
    grid=(HID // 128, NB),
    in_specs=[
        pl.BlockSpec((BN, 128), lambda c, i: (i, c)),
        pl.BlockSpec((2, 128), lambda c, i: (0, c)),
        pl.BlockSpec((1, 128), lambda c, i: (0, c)),
        pl.BlockSpec((1, 128), lambda c, i: (0, c)),
    ],
    out_specs=pl.BlockSpec((1, BN, 128), lambda c, i: (c, i, 0)),
    out_shape=jax.ShapeDtypeStruct((HID // 128, N, 128), jnp.float32),
)


def _head_body(h_ref, wo_ref, bo_ref, wm_ref, bm_ref, wl_ref, bl_ref,
               mu_ref, lv_ref):
    ne = jnp.broadcast_to(bo_ref[...], (BN, D_OUT))
    for c in range(HID // 128):
        ne = ne + jnp.dot(h_ref[c].astype(jnp.bfloat16),
                          wo_ref[c * 128:(c + 1) * 128, :].astype(jnp.bfloat16),
                          preferred_element_type=jnp.float32)
    ne16 = ne.astype(jnp.bfloat16)
    mu_ref[...] = jnp.dot(ne16, wm_ref[...].astype(jnp.bfloat16),
                          preferred_element_type=jnp.float32) + bm_ref[...]
    lv_ref[...] = jnp.dot(ne16, wl_ref[...].astype(jnp.bfloat16),
                          preferred_element_type=jnp.float32) + bl_ref[...]


_head = pl.pallas_call(
    _head_body,
    grid=(NB,),
    in_specs=[
        pl.BlockSpec((HID // 128, BN, 128), lambda i: (0, i, 0)),
        pl.BlockSpec((HID, D_OUT), lambda i: (0, 0)),
        pl.BlockSpec((1, D_OUT), lambda i: (0, 0)),
        pl.BlockSpec((D_OUT, D_VAE), lambda i: (0, 0)),
        pl.BlockSpec((1, D_VAE), lambda i: (0, 0)),
        pl.BlockSpec((D_OUT, D_VAE), lambda i: (0, 0)),
        pl.BlockSpec((1, D_VAE), lambda i: (0, 0)),
    ],
    out_specs=[
        pl.BlockSpec((BN, D_VAE), lambda i: (i, 0)),
        pl.BlockSpec((BN, D_VAE), lambda i: (i, 0)),
    ],
    out_shape=[
        jax.ShapeDtypeStruct((N, D_VAE), jnp.float32),
        jax.ShapeDtypeStruct((N, D_VAE), jnp.float32),
    ],
)


def kernel(x, edge_index, params):
    srcp = jnp.concatenate(
        [edge_index[0], jnp.zeros((PE - E,), jnp.int32)])
    dstp = jnp.concatenate(
        [edge_index[1], jnp.full((PE - E,), DUMMY, jnp.int32)])
    hcb = x.reshape(N, D_IN // 128, 128).transpose(1, 0, 2)

    C = D_IN // 128
    for l in range(L):
        parts = _make_sc_agg(C)(hcb, srcp, dstp)
        y, st = _make_mm(C)(
            params[f'eps_{l}'].reshape(1),
            hcb, parts,
            params[f'W1_{l}'], params[f'b1_{l}'].reshape(1, HID),
            params[f'W2_{l}'], params[f'b2_{l}'].reshape(1, HID))
        hcb = _bn_apply(y, st,
                        params[f'gamma_{l}'].reshape(1, HID),
                        params[f'beta_{l}'].reshape(1, HID))
        C = HID // 128

    mean, log_var = _head(
        hcb,
        params['W_out'], params['b_out'].reshape(1, D_OUT),
        params['W_mu'], params['b_mu'].reshape(1, D_VAE),
        params['W_lv'], params['b_lv'].reshape(1, D_VAE))
    return mean, log_var
